# Initial kernel scaffold; baseline (speedup 1.0000x reference)
import jax
import jax.numpy as jnp
from jax.experimental import pallas as pl


def kernel(x_ligand, x_protein, edge_index_lp, edge_index_pp, edge_index_ll, Wl_lp, bl_lp, Wr_lp, Wl_pp, bl_pp, Wr_pp, Wl_ll, bl_ll, Wr_ll, W_lin, b_lin):
    raise NotImplementedError("write your pallas kernel here")


# speedup vs baseline: 3.2541x; 3.2541x over previous
import functools

import jax
import jax.numpy as jnp
from jax import lax
from jax.experimental import pallas as pl
from jax.experimental.pallas import tpu as pltpu
from jax.experimental.pallas import tpu_sc as plsc

N = 10000
D = 128
E = 160000
NC = 2
NS = 16
NW = NC * NS
CHUNK = 64
EPW = 5120
NCHUNK = EPW // CHUNK
EPAD = NW * EPW
NPAD = 10240
ZROWS = NPAD // NS


def _sc_segment_sum(x, src3d, dst3d):
    mesh = plsc.VectorSubcoreMesh(core_axis_name="c", subcore_axis_name="s",
                                  num_cores=NC, num_subcores=NS)

    @functools.partial(
        pl.kernel,
        out_type=jax.ShapeDtypeStruct((NC, NPAD, D), jnp.float32),
        mesh=mesh,
        scratch_types=[
            pltpu.VMEM((NCHUNK, CHUNK), jnp.int32),
            pltpu.VMEM((NCHUNK, CHUNK), jnp.int32),
            pltpu.VMEM((CHUNK, D), jnp.float32),
            pltpu.VMEM_SHARED((NPAD, D), jnp.float32),
            pltpu.SemaphoreType.DMA,
        ],
    )
    def k(x_hbm, src_hbm, dst_hbm,
          out_acc, src_v, dst_v, gbuf, acc_sh, sem):
        c = lax.axis_index("c")
        s = lax.axis_index("s")
        wid = c * NS + s

        pltpu.sync_copy(src_hbm.at[wid], src_v)
        pltpu.sync_copy(dst_hbm.at[wid], dst_v)

        def fill(i, carry):
            r = i // (D // 16)
            l = (i % (D // 16)) * 16
            gbuf[r, pl.ds(l, 16)] = jnp.zeros((16,), jnp.float32)
            return carry

        lax.fori_loop(0, CHUNK * (D // 16), fill, 0)

        for t in range(ZROWS // CHUNK):
            pltpu.sync_copy(gbuf, acc_sh.at[pl.ds(s * ZROWS + t * CHUNK, CHUNK)])
        plsc.subcore_barrier()

        def chunk(j, carry):
            pltpu.async_copy(x_hbm.at[src_v.at[j]], gbuf, sem).wait()
            pltpu.sync_copy(gbuf, acc_sh.at[dst_v.at[j]], add=True)
            return carry

        lax.fori_loop(0, NCHUNK, chunk, 0)
        plsc.subcore_barrier()

        pltpu.sync_copy(acc_sh.at[pl.ds(s * ZROWS, ZROWS)],
                        out_acc.at[c].at[pl.ds(s * ZROWS, ZROWS)])

    return k(x, src3d, dst3d)


def _sc_segment_count(dst3d):
    mesh = plsc.VectorSubcoreMesh(core_axis_name="c", subcore_axis_name="s",
                                  num_cores=NC, num_subcores=NS)

    @functools.partial(
        pl.kernel,
        out_type=jax.ShapeDtypeStruct((NC, NPAD, D), jnp.float32),
        mesh=mesh,
        scratch_types=[
            pltpu.VMEM((NCHUNK, CHUNK), jnp.int32),
            pltpu.VMEM((CHUNK, D), jnp.float32),
            pltpu.VMEM((CHUNK, D), jnp.float32),
            pltpu.VMEM_SHARED((NPAD, D), jnp.float32),
        ],
    )
    def k(dst_hbm, out_cnt, dst_v, ones_v, zero_v, cnt_sh):
        c = lax.axis_index("c")
        s = lax.axis_index("s")
        wid = c * NS + s

        pltpu.sync_copy(dst_hbm.at[wid], dst_v)

        def fill1(i, carry):
            for q in range(D // 16):
                ones_v[i, pl.ds(q * 16, 16)] = jnp.ones((16,), jnp.float32)
                zero_v[i, pl.ds(q * 16, 16)] = jnp.zeros((16,), jnp.float32)
            return carry

        lax.fori_loop(0, CHUNK, fill1, 0)

        for t in range(ZROWS // CHUNK):
            pltpu.sync_copy(zero_v, cnt_sh.at[pl.ds(s * ZROWS + t * CHUNK, CHUNK)])
        plsc.subcore_barrier()

        def chunk(j, carry):
            pltpu.sync_copy(ones_v, cnt_sh.at[dst_v.at[j]], add=True)
            return carry

        lax.fori_loop(0, NCHUNK, chunk, 0)
        plsc.subcore_barrier()

        pltpu.sync_copy(cnt_sh.at[pl.ds(s * ZROWS, ZROWS)],
                        out_cnt.at[c].at[pl.ds(s * ZROWS, ZROWS)])

    return k(dst3d)


R = 2000
GRID = N // R


def _tc_body(x_ref, p0_ref, p1_ref, c0_ref, c1_ref, wl_ref, wr_ref, bl_ref,
             wlin_ref, blin_ref, o_ref, acc_ref):
    i = pl.program_id(0)

    @pl.when(i == 0)
    def _():
        acc_ref[...] = jnp.zeros_like(acc_ref)

    cnt = c0_ref[:, 0:1] + c1_ref[:, 0:1]
    mean = (p0_ref[...] + p1_ref[...]) / jnp.maximum(cnt, 1.0)
    out = (
        lax.dot_general(mean, wl_ref[...], (((1,), (1,)), ((), ())),
                        preferred_element_type=jnp.float32,
                        precision=lax.Precision.HIGHEST)
        + lax.dot_general(x_ref[...], wr_ref[...], (((1,), (1,)), ((), ())),
                          preferred_element_type=jnp.float32,
                          precision=lax.Precision.HIGHEST)
        + bl_ref[...]
    )
    h = jnp.maximum(out, 0.0)
    acc_ref[...] += jnp.sum(h.reshape(R // 8, 8, D), axis=0)

    @pl.when(i == GRID - 1)
    def _():
        colmean = jnp.sum(acc_ref[...], axis=0, keepdims=True) * (1.0 / N)
        o_ref[...] = jnp.sum(colmean * wlin_ref[...], axis=1, keepdims=True) \
            + blin_ref[...]


def _tc_finish(x, p0, p1, c0, c1, Wl, Wr, bl, Wlin, blin):
    out = pl.pallas_call(
        _tc_body,
        grid=(GRID,),
        in_specs=[
            pl.BlockSpec((R, D), lambda i: (i, 0)),
            pl.BlockSpec((R, D), lambda i: (i, 0)),
            pl.BlockSpec((R, D), lambda i: (i, 0)),
            pl.BlockSpec((R, 16), lambda i: (i, 0)),
            pl.BlockSpec((R, 16), lambda i: (i, 0)),
            pl.BlockSpec((D, D), lambda i: (0, 0)),
            pl.BlockSpec((D, D), lambda i: (0, 0)),
            pl.BlockSpec((1, D), lambda i: (0, 0)),
            pl.BlockSpec((1, D), lambda i: (0, 0)),
            pl.BlockSpec((1, 1), lambda i: (0, 0)),
        ],
        out_specs=pl.BlockSpec((1, 1), lambda i: (0, 0)),
        out_shape=jax.ShapeDtypeStruct((1, 1), jnp.float32),
        scratch_shapes=[pltpu.VMEM((8, D), jnp.float32)],
    )(x, p0, p1, c0, c1, Wl, Wr, bl, Wlin, blin)
    return out


def kernel(x_ligand, x_protein, edge_index_lp, edge_index_pp, edge_index_ll,
           Wl_lp, bl_lp, Wr_lp, Wl_pp, bl_pp, Wr_pp, Wl_ll, bl_ll, Wr_ll,
           W_lin, b_lin):
    src = edge_index_ll[0].astype(jnp.int32)
    dst = edge_index_ll[1].astype(jnp.int32)
    src = jnp.concatenate([src, jnp.zeros((EPAD - E,), jnp.int32)])
    dst = jnp.concatenate([dst, jnp.full((EPAD - E,), N, jnp.int32)])
    src3d = src.reshape(NW, NCHUNK, CHUNK)
    dst3d = dst.reshape(NW, NCHUNK, CHUNK)

    pacc = _sc_segment_sum(x_ligand, src3d, dst3d)
    pcnt = _sc_segment_count(dst3d)

    out = _tc_finish(x_ligand, pacc[0], pacc[1],
                     pcnt[0, :, :16], pcnt[1, :, :16],
                     Wl_ll, Wr_ll, bl_ll.reshape(1, D), W_lin,
                     b_lin.reshape(1, 1))
    return out.reshape(1)


# speedup vs baseline: 3.4682x; 1.0658x over previous
import functools

import jax
import jax.numpy as jnp
from jax import lax
from jax.experimental import pallas as pl
from jax.experimental.pallas import tpu as pltpu
from jax.experimental.pallas import tpu_sc as plsc

N = 10000
D = 128
E = 160000
NC = 2
NS = 16
NW = NC * NS
CHUNK = 64
EPW = 5120
NCHUNK = EPW // CHUNK
EPAD = NW * EPW
NPAD = 10240
ZROWS = NPAD // NS


def _sc_segment_sum(x, src3d, dst3d):
    mesh = plsc.VectorSubcoreMesh(core_axis_name="c", subcore_axis_name="s",
                                  num_cores=NC, num_subcores=NS)

    @functools.partial(
        pl.kernel,
        out_type=jax.ShapeDtypeStruct((NC, NPAD, D), jnp.float32),
        mesh=mesh,
        scratch_types=[
            pltpu.VMEM((NCHUNK, CHUNK), jnp.int32),
            pltpu.VMEM((NCHUNK, CHUNK), jnp.int32),
            pltpu.VMEM((CHUNK, D), jnp.float32),
            pltpu.VMEM((CHUNK, D), jnp.float32),
            pltpu.VMEM_SHARED((NPAD, D), jnp.float32),
            pltpu.SemaphoreType.DMA,
            pltpu.SemaphoreType.DMA,
        ],
    )
    def k(x_hbm, src_hbm, dst_hbm,
          out_acc, src_v, dst_v, gbuf, gbuf1, acc_sh, sem, sem1):
        c = lax.axis_index("c")
        s = lax.axis_index("s")
        wid = c * NS + s

        pltpu.sync_copy(src_hbm.at[wid], src_v)
        pltpu.sync_copy(dst_hbm.at[wid], dst_v)

        def fill(i, carry):
            r = i // (D // 16)
            l = (i % (D // 16)) * 16
            gbuf[r, pl.ds(l, 16)] = jnp.zeros((16,), jnp.float32)
            return carry

        lax.fori_loop(0, CHUNK * (D // 16), fill, 0)

        for t in range(ZROWS // CHUNK):
            pltpu.sync_copy(gbuf, acc_sh.at[pl.ds(s * ZROWS + t * CHUNK, CHUNK)])
        plsc.subcore_barrier()

        pltpu.async_copy(x_hbm.at[src_v.at[0]], gbuf, sem)

        def pair(h, carry):
            j = 2 * h
            pltpu.make_async_copy(x_hbm.at[src_v.at[j]], gbuf, sem).wait()
            pltpu.async_copy(x_hbm.at[src_v.at[j + 1]], gbuf1, sem1)
            pltpu.sync_copy(gbuf, acc_sh.at[dst_v.at[j]], add=True)
            pltpu.make_async_copy(x_hbm.at[src_v.at[j + 1]], gbuf1, sem1).wait()

            @pl.when(h < NCHUNK // 2 - 1)
            def _():
                pltpu.async_copy(x_hbm.at[src_v.at[j + 2]], gbuf, sem)

            pltpu.sync_copy(gbuf1, acc_sh.at[dst_v.at[j + 1]], add=True)
            return carry

        lax.fori_loop(0, NCHUNK // 2, pair, 0)
        plsc.subcore_barrier()

        pltpu.sync_copy(acc_sh.at[pl.ds(s * ZROWS, ZROWS)],
                        out_acc.at[c].at[pl.ds(s * ZROWS, ZROWS)])

    return k(x, src3d, dst3d)


def _sc_segment_count(dst3d):
    mesh = plsc.VectorSubcoreMesh(core_axis_name="c", subcore_axis_name="s",
                                  num_cores=NC, num_subcores=NS)

    @functools.partial(
        pl.kernel,
        out_type=jax.ShapeDtypeStruct((NC, NPAD, D), jnp.float32),
        mesh=mesh,
        scratch_types=[
            pltpu.VMEM((NCHUNK, CHUNK), jnp.int32),
            pltpu.VMEM((CHUNK, D), jnp.float32),
            pltpu.VMEM((CHUNK, D), jnp.float32),
            pltpu.VMEM_SHARED((NPAD, D), jnp.float32),
        ],
    )
    def k(dst_hbm, out_cnt, dst_v, ones_v, zero_v, cnt_sh):
        c = lax.axis_index("c")
        s = lax.axis_index("s")
        wid = c * NS + s

        pltpu.sync_copy(dst_hbm.at[wid], dst_v)

        def fill1(i, carry):
            for q in range(D // 16):
                ones_v[i, pl.ds(q * 16, 16)] = jnp.ones((16,), jnp.float32)
                zero_v[i, pl.ds(q * 16, 16)] = jnp.zeros((16,), jnp.float32)
            return carry

        lax.fori_loop(0, CHUNK, fill1, 0)

        for t in range(ZROWS // CHUNK):
            pltpu.sync_copy(zero_v, cnt_sh.at[pl.ds(s * ZROWS + t * CHUNK, CHUNK)])
        plsc.subcore_barrier()

        def chunk(j, carry):
            pltpu.sync_copy(ones_v, cnt_sh.at[dst_v.at[j]], add=True)
            return carry

        lax.fori_loop(0, NCHUNK, chunk, 0)
        plsc.subcore_barrier()

        pltpu.sync_copy(cnt_sh.at[pl.ds(s * ZROWS, ZROWS)],
                        out_cnt.at[c].at[pl.ds(s * ZROWS, ZROWS)])

    return k(dst3d)


R = 2000
GRID = N // R


def _tc_body(x_ref, p0_ref, p1_ref, c0_ref, c1_ref, wl_ref, wr_ref, bl_ref,
             wlin_ref, blin_ref, o_ref, acc_ref):
    i = pl.program_id(0)

    @pl.when(i == 0)
    def _():
        acc_ref[...] = jnp.zeros_like(acc_ref)

    cnt = c0_ref[:, 0:1] + c1_ref[:, 0:1]
    mean = (p0_ref[...] + p1_ref[...]) / jnp.maximum(cnt, 1.0)
    out = (
        lax.dot_general(mean, wl_ref[...], (((1,), (1,)), ((), ())),
                        preferred_element_type=jnp.float32,
                        precision=lax.Precision.HIGHEST)
        + lax.dot_general(x_ref[...], wr_ref[...], (((1,), (1,)), ((), ())),
                          preferred_element_type=jnp.float32,
                          precision=lax.Precision.HIGHEST)
        + bl_ref[...]
    )
    h = jnp.maximum(out, 0.0)
    acc_ref[...] += jnp.sum(h.reshape(R // 8, 8, D), axis=0)

    @pl.when(i == GRID - 1)
    def _():
        colmean = jnp.sum(acc_ref[...], axis=0, keepdims=True) * (1.0 / N)
        o_ref[...] = jnp.sum(colmean * wlin_ref[...], axis=1, keepdims=True) \
            + blin_ref[...]


def _tc_finish(x, p0, p1, c0, c1, Wl, Wr, bl, Wlin, blin):
    out = pl.pallas_call(
        _tc_body,
        grid=(GRID,),
        in_specs=[
            pl.BlockSpec((R, D), lambda i: (i, 0)),
            pl.BlockSpec((R, D), lambda i: (i, 0)),
            pl.BlockSpec((R, D), lambda i: (i, 0)),
            pl.BlockSpec((R, 16), lambda i: (i, 0)),
            pl.BlockSpec((R, 16), lambda i: (i, 0)),
            pl.BlockSpec((D, D), lambda i: (0, 0)),
            pl.BlockSpec((D, D), lambda i: (0, 0)),
            pl.BlockSpec((1, D), lambda i: (0, 0)),
            pl.BlockSpec((1, D), lambda i: (0, 0)),
            pl.BlockSpec((1, 1), lambda i: (0, 0)),
        ],
        out_specs=pl.BlockSpec((1, 1), lambda i: (0, 0)),
        out_shape=jax.ShapeDtypeStruct((1, 1), jnp.float32),
        scratch_shapes=[pltpu.VMEM((8, D), jnp.float32)],
    )(x, p0, p1, c0, c1, Wl, Wr, bl, Wlin, blin)
    return out


def kernel(x_ligand, x_protein, edge_index_lp, edge_index_pp, edge_index_ll,
           Wl_lp, bl_lp, Wr_lp, Wl_pp, bl_pp, Wr_pp, Wl_ll, bl_ll, Wr_ll,
           W_lin, b_lin):
    src = edge_index_ll[0].astype(jnp.int32)
    dst = edge_index_ll[1].astype(jnp.int32)
    src = jnp.concatenate([src, jnp.zeros((EPAD - E,), jnp.int32)])
    dst = jnp.concatenate([dst, jnp.full((EPAD - E,), N, jnp.int32)])
    src3d = src.reshape(NW, NCHUNK, CHUNK)
    dst3d = dst.reshape(NW, NCHUNK, CHUNK)

    pacc = _sc_segment_sum(x_ligand, src3d, dst3d)
    pcnt = _sc_segment_count(dst3d)

    out = _tc_finish(x_ligand, pacc[0], pacc[1],
                     pcnt[0, :, :16], pcnt[1, :, :16],
                     Wl_ll, Wr_ll, bl_ll.reshape(1, D), W_lin,
                     b_lin.reshape(1, 1))
    return out.reshape(1)


# speedup vs baseline: 4.0303x; 1.1621x over previous
import functools

import jax
import jax.numpy as jnp
from jax import lax
from jax.experimental import pallas as pl
from jax.experimental.pallas import tpu as pltpu
from jax.experimental.pallas import tpu_sc as plsc

N = 10000
D = 128
E = 160000
NC = 2
NS = 16
CHUNK = 128
NCHUNK = 80
EPW = NCHUNK * CHUNK
EPAD = NS * EPW
NPAD = 10240
ZROWS = NPAD // NS
NBUF = 2
HC = NCHUNK // 2


def _sc_sum_and_count(x, src3d, dst3d):
    mesh = plsc.VectorSubcoreMesh(core_axis_name="c", subcore_axis_name="s",
                                  num_cores=NC, num_subcores=NS)

    @functools.partial(
        pl.kernel,
        out_type=jax.ShapeDtypeStruct((NC, NPAD, D), jnp.float32),
        mesh=mesh,
        scratch_types=[
            pltpu.VMEM((HC, CHUNK), jnp.int32),
            pltpu.VMEM((HC, CHUNK), jnp.int32),
            [pltpu.VMEM((CHUNK, D), jnp.float32) for _ in range(NBUF)],
            pltpu.VMEM_SHARED((NPAD, D), jnp.float32),
            [pltpu.SemaphoreType.DMA for _ in range(NBUF)],
        ],
    )
    def k(x_hbm, src_hbm, dst_hbm,
          out, src_v, dst_v, gbufs, tbl_sh, sems):
        ones_v = gbufs[1]
        c = lax.axis_index("c")
        s = lax.axis_index("s")

        def fill(i, carry):
            r = i // (D // 16)
            l = (i % (D // 16)) * 16
            gbufs[0][r, pl.ds(l, 16)] = jnp.zeros((16,), jnp.float32)
            ones_v[r, pl.ds(l, 16)] = jnp.ones((16,), jnp.float32)
            return carry

        lax.fori_loop(0, CHUNK * (D // 16), fill, 0)

        for t in range(ZROWS // CHUNK):
            pltpu.sync_copy(gbufs[0], tbl_sh.at[pl.ds(s * ZROWS + t * CHUNK, CHUNK)])
        plsc.subcore_barrier()

        for half in range(2):
            pltpu.sync_copy(dst_hbm.at[s].at[pl.ds(half * HC, HC)], dst_v)

            @pl.when(c == 0)
            def _sums():
                pltpu.sync_copy(src_hbm.at[s].at[pl.ds(half * HC, HC)], src_v)
                for b in range(NBUF):
                    pltpu.async_copy(x_hbm.at[src_v.at[b]], gbufs[b], sems[b])

                def group(h, carry):
                    j = h * NBUF
                    for b in range(NBUF):
                        pltpu.make_async_copy(x_hbm.at[src_v.at[j + b]],
                                              gbufs[b], sems[b]).wait()
                        pltpu.sync_copy(gbufs[b], tbl_sh.at[dst_v.at[j + b]],
                                        add=True)

                        @pl.when(h < HC // NBUF - 1)
                        def _():
                            pltpu.async_copy(x_hbm.at[src_v.at[j + b + NBUF]],
                                             gbufs[b], sems[b])
                    return carry

                lax.fori_loop(0, HC // NBUF, group, 0)

            @pl.when(c == 1)
            def _counts():
                def chunk(j, carry):
                    pltpu.sync_copy(ones_v, tbl_sh.at[dst_v.at[j]], add=True)
                    return carry

                lax.fori_loop(0, HC, chunk, 0)

        plsc.subcore_barrier()

        for t in range(ZROWS // CHUNK):
            base = s * ZROWS + t * CHUNK
            pltpu.sync_copy(tbl_sh.at[pl.ds(base, CHUNK)],
                            out.at[c].at[pl.ds(base, CHUNK)])

    return k(x, src3d, dst3d)


R = 2000
GRID = N // R
DH = D // 2


def _tc_body(x_ref, p_ref, c_ref, wl_ref, wr_ref, bl_ref,
             wlin_ref, blin_ref, o_ref, acc_ref):
    i = pl.program_id(0)

    @pl.when(i == 0)
    def _():
        acc_ref[...] = jnp.zeros_like(acc_ref)

    cnt = jnp.maximum(c_ref[:, 0:1], 1.0)
    mean = p_ref[...] / cnt
    out = (
        lax.dot_general(mean, wl_ref[...], (((1,), (1,)), ((), ())),
                        preferred_element_type=jnp.float32,
                        precision=lax.Precision.HIGHEST)
        + lax.dot_general(x_ref[...], wr_ref[...], (((1,), (1,)), ((), ())),
                          preferred_element_type=jnp.float32,
                          precision=lax.Precision.HIGHEST)
        + bl_ref[...]
    )
    h = jnp.maximum(out, 0.0)
    acc_ref[...] += jnp.sum(h.reshape(R // 8, 8, D), axis=0)

    @pl.when(i == GRID - 1)
    def _():
        colmean = jnp.sum(acc_ref[...], axis=0, keepdims=True) * (1.0 / N)
        o_ref[...] = jnp.sum(colmean * wlin_ref[...], axis=1, keepdims=True) \
            + blin_ref[...]


def _tc_finish(x, p, cnt, Wl, Wr, bl, Wlin, blin):
    out = pl.pallas_call(
        _tc_body,
        grid=(GRID,),
        in_specs=[
            pl.BlockSpec((R, D), lambda i: (i, 0)),
            pl.BlockSpec((R, D), lambda i: (i, 0)),
            pl.BlockSpec((R, 16), lambda i: (i, 0)),
            pl.BlockSpec((D, D), lambda i: (0, 0)),
            pl.BlockSpec((D, D), lambda i: (0, 0)),
            pl.BlockSpec((1, D), lambda i: (0, 0)),
            pl.BlockSpec((1, D), lambda i: (0, 0)),
            pl.BlockSpec((1, 1), lambda i: (0, 0)),
        ],
        out_specs=pl.BlockSpec((1, 1), lambda i: (0, 0)),
        out_shape=jax.ShapeDtypeStruct((1, 1), jnp.float32),
        scratch_shapes=[pltpu.VMEM((8, D), jnp.float32)],
    )(x, p, cnt, Wl, Wr, bl, Wlin, blin)
    return out


def kernel(x_ligand, x_protein, edge_index_lp, edge_index_pp, edge_index_ll,
           Wl_lp, bl_lp, Wr_lp, Wl_pp, bl_pp, Wr_pp, Wl_ll, bl_ll, Wr_ll,
           W_lin, b_lin):
    src = edge_index_ll[0].astype(jnp.int32)
    dst = edge_index_ll[1].astype(jnp.int32)
    src = jnp.concatenate([src, jnp.zeros((EPAD - E,), jnp.int32)])
    dst = jnp.concatenate([dst, jnp.full((EPAD - E,), N, jnp.int32)])
    src3d = src.reshape(NS, NCHUNK, CHUNK)
    dst3d = dst.reshape(NS, NCHUNK, CHUNK)

    res = _sc_sum_and_count(x_ligand, src3d, dst3d)

    out = _tc_finish(x_ligand, res[0], res[1, :, :16],
                     Wl_ll, Wr_ll, bl_ll.reshape(1, D), W_lin,
                     b_lin.reshape(1, 1))
    return out.reshape(1)
